# baseline (speedup 1.0000x reference)
import functools

import jax
import jax.numpy as jnp
from jax import lax
from jax.experimental import pallas as pl
from jax.experimental.pallas import tpu as pltpu
from jax.experimental.pallas import tpu_sc as plsc

DIM = 128
ROWS = 10000
CH = 128
NCH = 200
NBUF = 3
NC = 2
NS = 16
NW = NC * NS
NROUNDS = NCH // NBUF
NTAIL = NCH - NROUNDS * NBUF


@functools.partial(
    pl.kernel,
    out_type=jax.ShapeDtypeStruct((NW * NCH * CH, DIM), jnp.float32),
    mesh=plsc.VectorSubcoreMesh(core_axis_name="c", subcore_axis_name="s"),
    scratch_types=[
        pltpu.VMEM((NBUF, CH), jnp.int32),
        pltpu.VMEM((NBUF, CH, DIM), jnp.float32),
        pltpu.VMEM_SHARED((ROWS, DIM), jnp.float32),
        pltpu.SemaphoreType.DMA((NBUF,)),
        pltpu.SemaphoreType.DMA((NBUF,)),
        pltpu.SemaphoreType.DMA((NBUF,)),
        pltpu.SemaphoreType.DMA,
    ],
)
def _sc_gather(x_hbm, table_hbm, out_hbm, idx_v, rows_v, table_sh, isem, gsem, ssem, tsem):
    wid = lax.axis_index("s") * NC + lax.axis_index("c")
    base = wid * (NCH * CH)

    sid = lax.axis_index("s")

    @pl.when(sid < 15)
    def _():
        pltpu.async_copy(
            table_hbm.at[pl.ds(sid * 624, 624)],
            table_sh.at[pl.ds(sid * 624, 624)],
            tsem,
        )

    @pl.when(sid == 15)
    def _():
        pltpu.async_copy(
            table_hbm.at[pl.ds(15 * 624, 640)],
            table_sh.at[pl.ds(15 * 624, 640)],
            tsem,
        )

    for b in range(NBUF):
        pltpu.async_copy(x_hbm.at[wid, b], idx_v.at[b], isem.at[b])

    for b in range(NBUF):
        pltpu.make_async_copy(x_hbm.at[wid, b], idx_v.at[b], isem.at[b]).wait()
        pltpu.async_copy(table_hbm.at[idx_v.at[b]], rows_v.at[b], gsem.at[b])

    @pl.when(sid < 15)
    def _():
        pltpu.make_async_copy(
            table_hbm.at[pl.ds(sid * 624, 624)],
            table_sh.at[pl.ds(sid * 624, 624)],
            tsem,
        ).wait()

    @pl.when(sid == 15)
    def _():
        pltpu.make_async_copy(
            table_hbm.at[pl.ds(15 * 624, 640)],
            table_sh.at[pl.ds(15 * 624, 640)],
            tsem,
        ).wait()

    plsc.subcore_barrier()

    @pl.loop(0, NROUNDS)
    def _round(g):
        for b in range(NBUF):
            j = g * NBUF + b
            pltpu.make_async_copy(
                table_sh.at[idx_v.at[b]], rows_v.at[b], gsem.at[b]
            ).wait()
            pltpu.async_copy(
                rows_v.at[b], out_hbm.at[pl.ds(base + j * CH, CH)], ssem.at[b]
            )

            bp = (b - 1) % NBUF
            jp = j - 1

            if b == 0:
                cond = (g > 0) & (jp + NBUF < NCH)
            else:
                cond = jp + NBUF < NCH

            @pl.when(cond)
            def _():
                pltpu.async_copy(
                    x_hbm.at[wid, jp + NBUF], idx_v.at[bp], isem.at[bp]
                )
                pltpu.make_async_copy(
                    rows_v.at[bp],
                    out_hbm.at[pl.ds(base + jp * CH, CH)],
                    ssem.at[bp],
                ).wait()
                pltpu.make_async_copy(
                    x_hbm.at[wid, jp + NBUF], idx_v.at[bp], isem.at[bp]
                ).wait()
                pltpu.async_copy(
                    table_sh.at[idx_v.at[bp]], rows_v.at[bp], gsem.at[bp]
                )


    for b in range(NTAIL):
        j = NROUNDS * NBUF + b
        pltpu.make_async_copy(
            table_sh.at[idx_v.at[b]], rows_v.at[b], gsem.at[b]
        ).wait()
        pltpu.async_copy(
            rows_v.at[b], out_hbm.at[pl.ds(base + j * CH, CH)], ssem.at[b]
        )

    for b in range(NBUF):
        j = NCH - NBUF + b
        pltpu.make_async_copy(
            rows_v.at[b], out_hbm.at[pl.ds(base + j * CH, CH)], ssem.at[b]
        ).wait()


def kernel(x, embedding):
    x2 = x.reshape(NW, NCH, CH)
    out = _sc_gather(x2, embedding)
    return out.reshape(x.shape[0], x.shape[1], DIM)


# speedup vs baseline: 1.0536x; 1.0536x over previous
import functools

import jax
import jax.numpy as jnp
from jax import lax
from jax.experimental import pallas as pl
from jax.experimental.pallas import tpu as pltpu
from jax.experimental.pallas import tpu_sc as plsc

DIM = 128
ROWS = 10000
CH = 128
NCH = 200
NBUF = 3
NC = 2
NS = 16
NW = NC * NS
NIDX = 2 * NBUF
NROUNDS = NCH // NIDX
NTAIL = NCH - NROUNDS * NIDX


@functools.partial(
    pl.kernel,
    out_type=jax.ShapeDtypeStruct((NW * NCH * CH, DIM), jnp.float32),
    mesh=plsc.VectorSubcoreMesh(core_axis_name="c", subcore_axis_name="s"),
    scratch_types=[
        pltpu.VMEM((NIDX, CH), jnp.int32),
        pltpu.VMEM((NBUF, CH, DIM), jnp.float32),
        pltpu.VMEM_SHARED((ROWS, DIM), jnp.float32),
        pltpu.SemaphoreType.DMA((NIDX,)),
        pltpu.SemaphoreType.DMA((NBUF,)),
        pltpu.SemaphoreType.DMA((NBUF,)),
        pltpu.SemaphoreType.DMA,
    ],
)
def _sc_gather(x_hbm, table_hbm, out_hbm, idx_v, rows_v, table_sh, isem, gsem, ssem, tsem):
    wid = lax.axis_index("s") * NC + lax.axis_index("c")
    base = wid * (NCH * CH)

    sid = lax.axis_index("s")

    @pl.when(sid < 15)
    def _():
        pltpu.async_copy(
            table_hbm.at[pl.ds(sid * 624, 624)],
            table_sh.at[pl.ds(sid * 624, 624)],
            tsem,
        )

    @pl.when(sid == 15)
    def _():
        pltpu.async_copy(
            table_hbm.at[pl.ds(15 * 624, 640)],
            table_sh.at[pl.ds(15 * 624, 640)],
            tsem,
        )

    for i in range(NIDX):
        pltpu.async_copy(x_hbm.at[wid, i], idx_v.at[i], isem.at[i])

    for b in range(NBUF):
        pltpu.make_async_copy(x_hbm.at[wid, b], idx_v.at[b], isem.at[b]).wait()
        pltpu.async_copy(table_hbm.at[idx_v.at[b]], rows_v.at[b], gsem.at[b])

    @pl.when(sid < 15)
    def _():
        pltpu.make_async_copy(
            table_hbm.at[pl.ds(sid * 624, 624)],
            table_sh.at[pl.ds(sid * 624, 624)],
            tsem,
        ).wait()

    @pl.when(sid == 15)
    def _():
        pltpu.make_async_copy(
            table_hbm.at[pl.ds(15 * 624, 640)],
            table_sh.at[pl.ds(15 * 624, 640)],
            tsem,
        ).wait()

    plsc.subcore_barrier()

    @pl.loop(0, NROUNDS)
    def _round(g):
        for i in range(NIDX):
            j = g * NIDX + i
            b = i % NBUF
            pltpu.make_async_copy(
                table_sh.at[idx_v.at[i]], rows_v.at[b], gsem.at[b]
            ).wait()
            pltpu.async_copy(
                rows_v.at[b], out_hbm.at[pl.ds(base + j * CH, CH)], ssem.at[b]
            )

            @pl.when(j + NIDX < NCH)
            def _():
                pltpu.async_copy(
                    x_hbm.at[wid, j + NIDX], idx_v.at[i], isem.at[i]
                )

            @pl.when(j + NBUF < NCH)
            def _():
                sn = (i + NBUF) % NIDX
                pltpu.make_async_copy(
                    rows_v.at[b],
                    out_hbm.at[pl.ds(base + j * CH, CH)],
                    ssem.at[b],
                ).wait()
                pltpu.make_async_copy(
                    x_hbm.at[wid, j + NBUF], idx_v.at[sn], isem.at[sn]
                ).wait()
                pltpu.async_copy(
                    table_sh.at[idx_v.at[sn]], rows_v.at[b], gsem.at[b]
                )

    for b in range(NTAIL):
        j = NROUNDS * NIDX + b
        pltpu.make_async_copy(
            table_sh.at[idx_v.at[b]], rows_v.at[b], gsem.at[b]
        ).wait()
        pltpu.async_copy(
            rows_v.at[b], out_hbm.at[pl.ds(base + j * CH, CH)], ssem.at[b]
        )

    for b in range(NBUF):
        j = NCH - NBUF + b
        pltpu.make_async_copy(
            rows_v.at[b], out_hbm.at[pl.ds(base + j * CH, CH)], ssem.at[b]
        ).wait()


def kernel(x, embedding):
    x2 = x.reshape(NW, NCH, CH)
    out = _sc_gather(x2, embedding)
    return out.reshape(x.shape[0], x.shape[1], DIM)
